# baseline (speedup 1.0000x reference)
import functools

import jax
import jax.numpy as jnp
from jax import lax
from jax.experimental import pallas as pl
from jax.experimental.pallas import tpu as pltpu
from jax.experimental.pallas import tpu_sc as plsc

_NUM_CORES = 2
_NUM_SUBCORES = 16
_NUM_WORKERS = _NUM_CORES * _NUM_SUBCORES

_CHUNK = 16


@functools.lru_cache(maxsize=None)
def _build(num_ids: int, d_model: int):
    assert num_ids % (_NUM_WORKERS * _CHUNK) == 0
    ids_per_worker = num_ids // _NUM_WORKERS
    n_chunks = ids_per_worker // _CHUNK

    mesh = plsc.VectorSubcoreMesh(core_axis_name="c", subcore_axis_name="s")

    @functools.partial(
        pl.kernel,
        mesh=mesh,
        out_type=jax.ShapeDtypeStruct((num_ids, d_model), jnp.float32),
        scratch_types=[
            pltpu.VMEM((ids_per_worker,), jnp.int32),
            pltpu.VMEM((_CHUNK, d_model), jnp.float32),
            pltpu.VMEM((_CHUNK, d_model), jnp.float32),
            pltpu.VMEM((_CHUNK, d_model), jnp.float32),
            pltpu.SemaphoreType.DMA,
            pltpu.SemaphoreType.DMA,
            pltpu.SemaphoreType.DMA,
            pltpu.SemaphoreType.DMA,
            pltpu.SemaphoreType.DMA,
            pltpu.SemaphoreType.DMA,
        ],
    )
    def gather_kernel(
        ids_hbm,
        table_hbm,
        out_hbm,
        idx_v,
        buf0,
        buf1,
        buf2,
        gsem0,
        gsem1,
        gsem2,
        ssem0,
        ssem1,
        ssem2,
    ):
        wid = lax.axis_index("s") * _NUM_CORES + lax.axis_index("c")
        base = wid * ids_per_worker
        pltpu.sync_copy(ids_hbm.at[pl.ds(base, ids_per_worker)], idx_v)

        bufs = (buf0, buf1, buf2)
        gsems = (gsem0, gsem1, gsem2)
        ssems = (ssem0, ssem1, ssem2)
        nbuf = len(bufs)

        def gather(i, b):
            return pltpu.make_async_copy(
                table_hbm.at[idx_v.at[pl.ds(i * _CHUNK, _CHUNK)]],
                bufs[b],
                gsems[b],
            )

        def scatter(i, b):
            return pltpu.make_async_copy(
                bufs[b],
                out_hbm.at[pl.ds(base + i * _CHUNK, _CHUNK)],
                ssems[b],
            )

        gather(0, 0).start()

        def step(k, bk, bn, first_round):
            nxt = k + 1
            if not first_round:
                scatter(nxt - nbuf, bn).wait()
            gather(nxt, bn).start()
            gather(k, bk).wait()
            scatter(k, bk).start()

        for k in range(nbuf - 1):
            step(k, k, k + 1, True)

        def body(t, carry):
            for j in range(nbuf):
                k = nbuf * t + (nbuf - 1) + j
                step(k, (nbuf - 1 + j) % nbuf, (nbuf + j) % nbuf, False)
            return carry

        n_loop = (n_chunks - 1 - (nbuf - 1)) // nbuf
        lax.fori_loop(0, n_loop, body, 0, unroll=False)

        for k in range((nbuf - 1) + n_loop * nbuf, n_chunks - 1):
            step(k, k % nbuf, (k + 1) % nbuf, False)
        last = n_chunks - 1
        gather(last, last % nbuf).wait()
        scatter(last, last % nbuf).start()

        for i in range(n_chunks - nbuf, n_chunks):
            scatter(i, i % nbuf).wait()

    return gather_kernel


def kernel(input_ids, embed_table):
    num_ids = input_ids.shape[0] * input_ids.shape[1]
    d_model = embed_table.shape[1]
    ids = input_ids.reshape(num_ids).astype(jnp.int32)
    out = _build(num_ids, d_model)(ids, embed_table)
    return out.reshape(input_ids.shape + (d_model,))


# speedup vs baseline: 1.0002x; 1.0002x over previous
import functools

import jax
import jax.numpy as jnp
from jax import lax
from jax.experimental import pallas as pl
from jax.experimental.pallas import tpu as pltpu
from jax.experimental.pallas import tpu_sc as plsc

_NUM_CORES = 2
_NUM_SUBCORES = 16
_NUM_WORKERS = _NUM_CORES * _NUM_SUBCORES

_CHUNK = 24
_NBUF = 2


@functools.lru_cache(maxsize=None)
def _build(num_ids: int, d_model: int):
    assert num_ids % (_NUM_WORKERS * 8) == 0
    ids_per_worker = num_ids // _NUM_WORKERS
    n_full = ids_per_worker // _CHUNK
    rem = ids_per_worker - n_full * _CHUNK
    assert rem % 8 == 0
    n_chunks = n_full + (1 if rem else 0)

    def rows_of(i):
        return _CHUNK if i < n_full else rem

    mesh = plsc.VectorSubcoreMesh(core_axis_name="c", subcore_axis_name="s")

    @functools.partial(
        pl.kernel,
        mesh=mesh,
        out_type=jax.ShapeDtypeStruct((num_ids, d_model), jnp.float32),
        scratch_types=[
            pltpu.VMEM((ids_per_worker,), jnp.int32),
            pltpu.VMEM((_CHUNK, d_model), jnp.float32),
            pltpu.VMEM((_CHUNK, d_model), jnp.float32),
            pltpu.SemaphoreType.DMA,
            pltpu.SemaphoreType.DMA,
            pltpu.SemaphoreType.DMA,
            pltpu.SemaphoreType.DMA,
        ],
    )
    def gather_kernel(
        ids_hbm, table_hbm, out_hbm, idx_v, buf0, buf1, gsem0, gsem1, ssem0, ssem1
    ):
        wid = lax.axis_index("s") * _NUM_CORES + lax.axis_index("c")
        base = wid * ids_per_worker
        pltpu.sync_copy(ids_hbm.at[pl.ds(base, ids_per_worker)], idx_v)

        bufs = (buf0, buf1)
        gsems = (gsem0, gsem1)
        ssems = (ssem0, ssem1)

        def gather(i, b, rows=_CHUNK):
            dst = bufs[b] if rows == _CHUNK else bufs[b].at[pl.ds(0, rows)]
            return pltpu.make_async_copy(
                table_hbm.at[idx_v.at[pl.ds(i * _CHUNK, rows)]], dst, gsems[b]
            )

        def scatter(i, b, rows=_CHUNK):
            src = bufs[b] if rows == _CHUNK else bufs[b].at[pl.ds(0, rows)]
            return pltpu.make_async_copy(
                src, out_hbm.at[pl.ds(base + i * _CHUNK, rows)], ssems[b]
            )

        gather(0, 0).start()

        def step(k, bk, bn, first_round, rows_next=_CHUNK):
            nxt = k + 1
            if not first_round:
                scatter(nxt - _NBUF, bn).wait()
            gather(nxt, bn, rows_next).start()
            gather(k, bk).wait()
            scatter(k, bk).start()

        step(0, 0, 1, True)

        n_loop = (n_chunks - 2 - (_NBUF - 1)) // _NBUF

        def body(t, carry):
            for j in range(_NBUF):
                k = _NBUF * t + 1 + j
                step(k, (1 + j) % _NBUF, j % _NBUF, False)
            return carry

        lax.fori_loop(0, n_loop, body, 0, unroll=False)

        for k in range(1 + n_loop * _NBUF, n_chunks - 1):
            step(k, k % _NBUF, (k + 1) % _NBUF, False, rows_of(k + 1))
        last = n_chunks - 1
        gather(last, last % _NBUF, rows_of(last)).wait()
        scatter(last, last % _NBUF, rows_of(last)).start()

        for i in range(n_chunks - _NBUF, n_chunks):
            scatter(i, i % _NBUF, rows_of(i)).wait()

    return gather_kernel


def kernel(input_ids, embed_table):
    num_ids = input_ids.shape[0] * input_ids.shape[1]
    d_model = embed_table.shape[1]
    ids = input_ids.reshape(num_ids).astype(jnp.int32)
    out = _build(num_ids, d_model)(ids, embed_table)
    return out.reshape(input_ids.shape + (d_model,))
